# baseline (speedup 1.0000x reference)
import functools

import jax
import jax.numpy as jnp
from jax import lax
from jax.experimental import pallas as pl
from jax.experimental.pallas import tpu as pltpu
from jax.experimental.pallas import tpu_sc as plsc

P = 8
N = 512
B = 64
EPS = 1e-06

NC = 2
NS = 16
NW = NC * NS
RB = N // NW


def _norm_body(s_ref, g_ref, m_ref):
    s = s_ref[0]
    rows = lax.broadcasted_iota(jnp.int32, (N, N), 0)
    cols = lax.broadcasted_iota(jnp.int32, (N, N), 1)
    sz = jnp.where(rows == cols, 0.0, s)
    denom = jnp.clip(jnp.sum(jnp.abs(sz), axis=1, keepdims=True), EPS, None)
    graw = g_ref[0]
    g = jnp.maximum(graw, 0.0) + jnp.log1p(jnp.exp(-jnp.abs(graw))) + 1e-06
    gsum = jnp.clip(jnp.sum(g), EPS, None)
    m_ref[0] = sz * (g * (N / gsum) / denom)


_mesh = plsc.VectorSubcoreMesh(core_axis_name="c", subcore_axis_name="s")


@functools.partial(
    pl.kernel,
    mesh=_mesh,
    out_type=jax.ShapeDtypeStruct((B, N, N), jnp.float32),
    scratch_types=[
        pltpu.VMEM((P, RB, N), jnp.float32),
        pltpu.VMEM((B + 16, ), jnp.int32),
        pltpu.SemaphoreType.DMA,
        pltpu.SemaphoreType.DMA,
    ],
)
def _sc_gather(m_hbm, ph_hbm, out_hbm, buf, ph_v, rsem, wsem):
    wid = lax.axis_index("s") * NC + lax.axis_index("c")
    r0 = wid * RB

    pltpu.sync_copy(ph_hbm, ph_v.at[pl.ds(0, B)])

    for p in range(P):
        pltpu.make_async_copy(
            m_hbm.at[p, pl.ds(r0, RB), :], buf.at[p], rsem
        ).start()
    for _ in range(P):
        pltpu.make_async_copy(
            m_hbm.at[0, pl.ds(r0, RB), :], buf.at[0], rsem
        ).wait()

    def issue(b, c):
        ph_b = ph_v[pl.ds(b, 16)][0]
        pltpu.make_async_copy(
            buf.at[ph_b], out_hbm.at[b, pl.ds(r0, RB), :], wsem
        ).start()
        return c

    lax.fori_loop(0, B, issue, 0)

    def drain(b, c):
        pltpu.make_async_copy(
            buf.at[0], out_hbm.at[0, pl.ds(r0, RB), :], wsem
        ).wait()
        return c

    lax.fori_loop(0, B, drain, 0)


@jax.jit
def kernel(phases, S, G):
    phases = phases.astype(jnp.int32)
    Gc = G.reshape(P, N, 1)

    M = pl.pallas_call(
        _norm_body,
        grid=(P,),
        in_specs=[
            pl.BlockSpec((1, N, N), lambda i: (i, 0, 0)),
            pl.BlockSpec((1, N, 1), lambda i: (i, 0, 0)),
        ],
        out_specs=pl.BlockSpec((1, N, N), lambda i: (i, 0, 0)),
        out_shape=jax.ShapeDtypeStruct((P, N, N), jnp.float32),
    )(S, Gc)

    return _sc_gather(M, phases)
